# Initial kernel scaffold; baseline (speedup 1.0000x reference)
import jax
import jax.numpy as jnp
from jax.experimental import pallas as pl


def kernel(prompt, logits):
    raise NotImplementedError("write your pallas kernel here")


# speedup vs baseline: 1.2013x; 1.2013x over previous
import functools

import jax
import jax.numpy as jnp
import numpy as np
from jax import lax
from jax.experimental import pallas as pl
from jax.experimental.pallas import tpu as pltpu
from jax.experimental.pallas import tpu_sc as plsc

B = 8
K = 8
T = 512
CARD = 2048
SPECIAL = 2048
S_OUT = T + K
S_IN = T + K - 1
NBLK = B * K
CH = 16
NCH = T // CH
NC = 2
NS = 16
NW = NC * NS
BLOCKS_PER_W = NBLK // NW


def _seq_mask_np():
    m = np.zeros((K, S_OUT), dtype=bool)
    for q in range(K):
        m[q, q + 1:q + 1 + T] = True
    return m


_SEQ_MASK = _seq_mask_np()


def _sc_copy_body(lg, out, buf0, buf1, sem0, sem1):
    wid = lax.axis_index("s") * NC + lax.axis_index("c")
    for blk in range(BLOCKS_PER_W):
        i = wid * BLOCKS_PER_W + blk
        q = lax.rem(i, K)
        src0 = pl.multiple_of(i * S_IN + q, 8)
        dst0 = pl.multiple_of(i * T, 8)

        def body(c, carry):
            src = pl.multiple_of(src0 + c * CH, 8)
            dst = pl.multiple_of(dst0 + c * CH, 8)
            pltpu.async_copy(lg.at[pl.ds(src, CH)], buf0, sem0).wait()
            pltpu.sync_copy(buf0, out.at[pl.ds(dst, CH)])
            return carry

        lax.fori_loop(0, NCH, body, 0)


def _sc_copy(logits_flat):
    mesh = plsc.VectorSubcoreMesh(core_axis_name="c", subcore_axis_name="s")
    fn = functools.partial(
        pl.kernel,
        mesh=mesh,
        out_type=jax.ShapeDtypeStruct((NBLK * T, CARD), jnp.float32),
        scratch_types=[
            pltpu.VMEM((CH, CARD), jnp.float32),
            pltpu.VMEM((CH, CARD), jnp.float32),
            pltpu.SemaphoreType.DMA,
            pltpu.SemaphoreType.DMA,
        ],
    )(_sc_copy_body)
    return fn(logits_flat)


def _values_body(p_ref, o_ref):
    for q in range(K):
        row = p_ref[:, q, :]
        parts = [jnp.full((B, q + 1), SPECIAL, jnp.int32), row]
        if K - 1 - q:
            parts.append(jnp.full((B, K - 1 - q), SPECIAL, jnp.int32))
        o_ref[:, q, :] = jnp.concatenate(parts, axis=1)


def _values(prompt):
    return pl.pallas_call(
        _values_body,
        out_shape=jax.ShapeDtypeStruct((B, K, S_OUT), jnp.int32),
    )(prompt)


def kernel(prompt, logits):
    lg_flat = logits.reshape(NBLK * S_IN, CARD)
    out_flat = _sc_copy(lg_flat)
    out_logits = out_flat.reshape(B, K, T, CARD)
    values = _values(prompt)
    seq_mask = jnp.asarray(_SEQ_MASK)
    logits_mask = jnp.ones((B, K, T), dtype=bool)
    return values, seq_mask, out_logits, logits_mask


# speedup vs baseline: 1.2762x; 1.0624x over previous
import functools

import jax
import jax.numpy as jnp
import numpy as np
from jax import lax
from jax.experimental import pallas as pl
from jax.experimental.pallas import tpu as pltpu
from jax.experimental.pallas import tpu_sc as plsc

B = 8
K = 8
T = 512
CARD = 2048
SPECIAL = 2048
S_OUT = T + K
S_IN = T + K - 1
NBLK = B * K
CH = 16
NCH = T // CH
NC = 2
NS = 16
NW = NC * NS
BLOCKS_PER_W = NBLK // NW


def _seq_mask_np():
    m = np.zeros((K, S_OUT), dtype=bool)
    for q in range(K):
        m[q, q + 1:q + 1 + T] = True
    return m


_SEQ_MASK = _seq_mask_np()


def _sc_copy_body(lg, out, buf0, buf1, rs0, rs1, ws0, ws1):
    wid = lax.axis_index("s") * NC + lax.axis_index("c")
    nchunks = BLOCKS_PER_W * NCH
    bufs, rsems, wsems = (buf0, buf1), (rs0, rs1), (ws0, ws1)

    def src_of(c):
        i = wid * BLOCKS_PER_W + lax.div(c, NCH)
        q = lax.rem(i, K)
        return pl.multiple_of(i * S_IN + q + lax.rem(c, NCH) * CH, 8)

    def dst_of(c):
        return pl.multiple_of(wid * BLOCKS_PER_W * T + c * CH, 8)

    def read_start(c, p):
        pltpu.make_async_copy(lg.at[pl.ds(src_of(c), CH)], bufs[p], rsems[p]).start()

    def read_wait(p):
        pltpu.make_async_copy(lg.at[pl.ds(0, CH)], bufs[p], rsems[p]).wait()

    def write_start(c, p):
        pltpu.make_async_copy(bufs[p], out.at[pl.ds(dst_of(c), CH)], wsems[p]).start()

    def write_wait(p):
        pltpu.make_async_copy(bufs[p], out.at[pl.ds(0, CH)], wsems[p]).wait()

    read_start(0, 0)
    read_start(1, 1)

    def body(cc, carry):
        c = cc * 2
        for p in range(2):
            read_wait(p)
            write_start(c + p, p)
        for p in range(2):
            write_wait(p)

            @pl.when(c + 2 + p < nchunks)
            def _():
                read_start(c + 2 + p, p)

        return carry

    lax.fori_loop(0, nchunks // 2, body, 0)


def _sc_copy(logits_flat):
    mesh = plsc.VectorSubcoreMesh(core_axis_name="c", subcore_axis_name="s")
    fn = functools.partial(
        pl.kernel,
        mesh=mesh,
        out_type=jax.ShapeDtypeStruct((NBLK * T, CARD), jnp.float32),
        scratch_types=[
            pltpu.VMEM((CH, CARD), jnp.float32),
            pltpu.VMEM((CH, CARD), jnp.float32),
            pltpu.SemaphoreType.DMA,
            pltpu.SemaphoreType.DMA,
            pltpu.SemaphoreType.DMA,
            pltpu.SemaphoreType.DMA,
        ],
    )(_sc_copy_body)
    return fn(logits_flat)


def _values_body(p_ref, o_ref):
    for q in range(K):
        row = p_ref[:, q, :]
        parts = [jnp.full((B, q + 1), SPECIAL, jnp.int32), row]
        if K - 1 - q:
            parts.append(jnp.full((B, K - 1 - q), SPECIAL, jnp.int32))
        o_ref[:, q, :] = jnp.concatenate(parts, axis=1)


def _values(prompt):
    return pl.pallas_call(
        _values_body,
        out_shape=jax.ShapeDtypeStruct((B, K, S_OUT), jnp.int32),
    )(prompt)


def kernel(prompt, logits):
    lg_flat = logits.reshape(NBLK * S_IN, CARD)
    out_flat = _sc_copy(lg_flat)
    out_logits = out_flat.reshape(B, K, T, CARD)
    values = _values(prompt)
    seq_mask = jnp.asarray(_SEQ_MASK)
    logits_mask = jnp.ones((B, K, T), dtype=bool)
    return values, seq_mask, out_logits, logits_mask
